# baseline (speedup 1.0000x reference)
import functools

import jax
import jax.numpy as jnp
from jax.experimental import pallas as pl
from jax.experimental.pallas import tpu as pltpu

B, S, D = 2, 2048, 1024
E, K = 8, 2
N = B * S
TILE_T = 512
NT = N // TILE_T


def _moe_dense_kernel(x_ref, xb_ref, wg_ref, wb_ref, o_ref, gate_scr):
    e = pl.program_id(1)

    @pl.when(e == 0)
    def _():
        logits = jax.lax.dot_general(
            x_ref[...], wg_ref[...], (((1,), (1,)), ((), ())),
            preferred_element_type=jnp.float32)
        iota = jax.lax.broadcasted_iota(jnp.int32, logits.shape, 1)
        m1 = jnp.max(logits, axis=1, keepdims=True)
        i1 = jnp.min(jnp.where(logits == m1, iota, E), axis=1, keepdims=True)
        masked = jnp.where(iota == i1, -jnp.inf, logits)
        m2 = jnp.max(masked, axis=1, keepdims=True)
        i2 = jnp.min(jnp.where(masked == m2, iota, E), axis=1, keepdims=True)
        z = jnp.exp(m2 - m1)
        w1 = 1.0 / (1.0 + z)
        w2 = z / (1.0 + z)
        gate_scr[...] = jnp.where(iota == i1, w1, 0.0) + jnp.where(iota == i2, w2, 0.0)
        o_ref[...] = jnp.zeros_like(o_ref)

    y = jax.lax.dot_general(
        xb_ref[...], wb_ref[0], (((1,), (1,)), ((), ())),
        preferred_element_type=jnp.float32)
    g = gate_scr[...]
    lane = jax.lax.broadcasted_iota(jnp.int32, g.shape, 1)
    gate_col = jnp.sum(jnp.where(lane == e, g, 0.0), axis=1, keepdims=True)
    o_ref[...] += gate_col * y


@jax.jit
def kernel(x, Wg, Wexp):
    xf = x.reshape(N, D)
    xb = xf.astype(jnp.bfloat16)
    wb = Wexp.astype(jnp.bfloat16)
    out = pl.pallas_call(
        _moe_dense_kernel,
        grid=(NT, E),
        in_specs=[
            pl.BlockSpec((TILE_T, D), lambda t, e: (t, 0)),
            pl.BlockSpec((TILE_T, D), lambda t, e: (t, 0)),
            pl.BlockSpec((E, D), lambda t, e: (0, 0)),
            pl.BlockSpec((1, D, D), lambda t, e: (e, 0, 0)),
        ],
        out_specs=pl.BlockSpec((TILE_T, D), lambda t, e: (t, 0)),
        out_shape=jax.ShapeDtypeStruct((N, D), jnp.float32),
        scratch_shapes=[pltpu.VMEM((TILE_T, E), jnp.float32)],
    )(xf, xb, Wg, wb)
    return out.reshape(B, S, D)
